# Initial kernel scaffold; baseline (speedup 1.0000x reference)
import functools

import jax
import jax.numpy as jnp
from jax import lax
from jax.experimental import pallas as pl
from jax.experimental.pallas import tpu as pltpu
from jax.experimental.pallas import tpu_sc as plsc

B, T, V = 1024, 50, 1000
NC, NS = 2, 16
NW = NC * NS
ROWS = B * T
RPW = ROWS // NW
CHUNK = 40
NCHUNK = RPW // CHUNK

_mesh = plsc.VectorSubcoreMesh(core_axis_name="c", subcore_axis_name="s")


@functools.partial(
    pl.kernel,
    out_type=jax.ShapeDtypeStruct((ROWS, V), jnp.float32),
    mesh=_mesh,
    scratch_types=[
        pltpu.VMEM((NCHUNK, CHUNK), jnp.int32),
        pltpu.VMEM((CHUNK, V), jnp.float32),
        pltpu.SemaphoreType.DMA,
    ],
)
def _sc_gather(table_hbm, idx_hbm, out_hbm, idx_v, rows_v, sem):
    wid = lax.axis_index("s") * NC + lax.axis_index("c")
    base = wid * RPW
    pltpu.sync_copy(idx_hbm.at[wid], idx_v)

    def body(c, carry):
        pltpu.async_copy(table_hbm.at[idx_v.at[c]], rows_v, sem).wait()
        pltpu.sync_copy(rows_v, out_hbm.at[pl.ds(base + c * CHUNK, CHUNK)])
        return carry

    lax.fori_loop(0, NCHUNK, body, 0)


def kernel(idx, log_probs):
    idx = idx.astype(jnp.int32)
    table = jnp.concatenate(
        [log_probs, jnp.zeros((1, V), log_probs.dtype)], axis=0
    )
    prev = jnp.concatenate(
        [jnp.full((B, 1), V, jnp.int32), idx[:, :-1]], axis=1
    )
    idx_full = prev.reshape(NW, NCHUNK, CHUNK)
    out = _sc_gather(table, idx_full)
    return out.reshape(B, T, V)


# speedup vs baseline: 1.1110x; 1.1110x over previous
import functools

import jax
import jax.numpy as jnp
from jax import lax
from jax.experimental import pallas as pl
from jax.experimental.pallas import tpu as pltpu
from jax.experimental.pallas import tpu_sc as plsc

B, T, V = 1024, 50, 1000
NC, NS = 2, 16
NW = NC * NS
ROWS = B * T
RPW = ROWS // NW
CHUNK = 40
NCHUNK = RPW // CHUNK

_mesh = plsc.VectorSubcoreMesh(core_axis_name="c", subcore_axis_name="s")


@functools.partial(
    pl.kernel,
    out_type=jax.ShapeDtypeStruct((ROWS, V), jnp.float32),
    mesh=_mesh,
    scratch_types=[
        pltpu.VMEM((NCHUNK, CHUNK), jnp.int32),
        pltpu.VMEM((CHUNK, V), jnp.float32),
        pltpu.SemaphoreType.DMA,
    ],
    compiler_params=pltpu.CompilerParams(use_tc_tiling_on_sc=False),
)
def _sc_gather(table_hbm, idx_hbm, out_hbm, idx_v, rows_v, sem):
    wid = lax.axis_index("s") * NC + lax.axis_index("c")
    base = wid * RPW
    pltpu.sync_copy(idx_hbm.at[wid], idx_v)

    def body(c, carry):
        pltpu.async_copy(table_hbm.at[idx_v.at[c]], rows_v, sem).wait()
        pltpu.sync_copy(rows_v, out_hbm.at[pl.ds(base + c * CHUNK, CHUNK)])
        return carry

    lax.fori_loop(0, NCHUNK, body, 0)


def kernel(idx, log_probs):
    idx = idx.astype(jnp.int32)
    table = jnp.concatenate(
        [log_probs, jnp.zeros((1, V), log_probs.dtype)], axis=0
    )
    prev = jnp.concatenate(
        [jnp.full((B, 1), V, jnp.int32), idx[:, :-1]], axis=1
    )
    idx_full = prev.reshape(NW, NCHUNK, CHUNK)
    out = _sc_gather(table, idx_full)
    return out.reshape(B, T, V)


# speedup vs baseline: 1.1146x; 1.0033x over previous
import functools

import jax
import jax.numpy as jnp
from jax import lax
from jax.experimental import pallas as pl
from jax.experimental.pallas import tpu as pltpu
from jax.experimental.pallas import tpu_sc as plsc

B, T, V = 1024, 50, 1000
NC, NS = 2, 16
NW = NC * NS
ROWS = B * T
RPW = ROWS // NW
CHUNK = 40
NCHUNK = RPW // CHUNK

_mesh = plsc.VectorSubcoreMesh(core_axis_name="c", subcore_axis_name="s")


@functools.partial(
    pl.kernel,
    out_type=jax.ShapeDtypeStruct((ROWS, V), jnp.float32),
    mesh=_mesh,
    scratch_types=[
        pltpu.VMEM((NCHUNK, CHUNK), jnp.int32),
        pltpu.VMEM((CHUNK, V), jnp.float32),
        pltpu.VMEM((CHUNK, V), jnp.float32),
        pltpu.SemaphoreType.DMA,
        pltpu.SemaphoreType.DMA,
    ],
    compiler_params=pltpu.CompilerParams(use_tc_tiling_on_sc=False),
)
def _sc_gather(table_hbm, idx_hbm, out_hbm, idx_v, rows_a, rows_b, sem_a, sem_b):
    wid = lax.axis_index("s") * NC + lax.axis_index("c")
    base = wid * RPW
    pltpu.sync_copy(idx_hbm.at[wid], idx_v)

    def gather(c, buf, sem):
        return pltpu.make_async_copy(table_hbm.at[idx_v.at[c]], buf, sem)

    gather(0, rows_a, sem_a).start()
    gather(1, rows_b, sem_b).start()

    def body(i, carry):
        c = 2 * i
        gather(c, rows_a, sem_a).wait()
        pltpu.sync_copy(rows_a, out_hbm.at[pl.ds(base + c * CHUNK, CHUNK)])
        gather(c + 2, rows_a, sem_a).start()
        gather(c + 1, rows_b, sem_b).wait()
        pltpu.sync_copy(rows_b, out_hbm.at[pl.ds(base + (c + 1) * CHUNK, CHUNK)])
        gather(c + 3, rows_b, sem_b).start()
        return carry

    lax.fori_loop(0, NCHUNK // 2 - 1, body, 0)

    c_last = NCHUNK - 2
    gather(c_last, rows_a, sem_a).wait()
    pltpu.sync_copy(rows_a, out_hbm.at[pl.ds(base + c_last * CHUNK, CHUNK)])
    gather(c_last + 1, rows_b, sem_b).wait()
    pltpu.sync_copy(rows_b, out_hbm.at[pl.ds(base + (c_last + 1) * CHUNK, CHUNK)])


def kernel(idx, log_probs):
    idx = idx.astype(jnp.int32)
    table = jnp.concatenate(
        [log_probs, jnp.zeros((1, V), log_probs.dtype)], axis=0
    )
    prev = jnp.concatenate(
        [jnp.full((B, 1), V, jnp.int32), idx[:, :-1]], axis=1
    )
    idx_full = prev.reshape(NW, NCHUNK, CHUNK)
    out = _sc_gather(table, idx_full)
    return out.reshape(B, T, V)
